# baseline (speedup 1.0000x reference)
import jax
import jax.numpy as jnp
from jax import lax
from jax.experimental import pallas as pl
from jax.experimental.pallas import tpu as pltpu

B, C, H, W = 8, 192, 224, 224
HW = H * W
ROWS = B * C
C2 = 2 * C
TOPK = 64
NPLANES = B * TOPK

_RB = 128
_CB = 7168
_GR = ROWS // _RB
_GC = HW // _CB


def _pool_body(x0_ref, x1_ref, s0_ref, s1_ref):
    j = pl.program_id(1)

    @pl.when(j == 0)
    def _():
        s0_ref[...] = jnp.zeros_like(s0_ref)
        s1_ref[...] = jnp.zeros_like(s1_ref)

    s0_ref[0, 0, :] += jnp.sum(x0_ref[...], axis=1)
    s1_ref[0, 0, :] += jnp.sum(x1_ref[...], axis=1)


def _topk_body(s0_ref, s1_ref, r0_ref, r1_ref, u0_ref):
    vals = jnp.concatenate([s0_ref[...], s1_ref[...]], axis=1) / float(HW)
    iota_c = lax.broadcasted_iota(jnp.int32, (B, C2), 1)
    iota_k = lax.broadcasted_iota(jnp.int32, (B, TOPK), 1)
    idxm = jnp.zeros((B, TOPK), jnp.int32)
    for k in range(TOPK):
        m = jnp.max(vals, axis=1, keepdims=True)
        cand = jnp.where(vals == m, iota_c, jnp.int32(2**30))
        sel = jnp.min(cand, axis=1)
        idxm = jnp.where(iota_k == k, sel[:, None], idxm)
        vals = jnp.where(iota_c == sel[:, None], -jnp.inf, vals)
    rowbase = lax.broadcasted_iota(jnp.int32, (B, TOPK), 0) * C
    r0_ref[...] = rowbase + jnp.minimum(idxm, C - 1)
    r1_ref[...] = rowbase + jnp.maximum(idxm - C, 0)
    u0_ref[...] = (idxm < C).astype(jnp.int32)


def _gather_body(r0s, r1s, u0s, a_ref, b_ref, o_ref):
    del r0s, r1s
    i = pl.program_id(0)
    o_ref[...] = jnp.where(u0s[i] == 1, a_ref[...], b_ref[...])


def kernel(x_0, x_1):
    x0r = x_0.reshape(ROWS, HW)
    x1r = x_1.reshape(ROWS, HW)

    s0, s1 = pl.pallas_call(
        _pool_body,
        grid=(_GR, _GC),
        in_specs=[pl.BlockSpec((_RB, _CB), lambda i, j: (i, j)),
                  pl.BlockSpec((_RB, _CB), lambda i, j: (i, j))],
        out_specs=[pl.BlockSpec((1, 1, _RB), lambda i, j: (i, 0, 0)),
                   pl.BlockSpec((1, 1, _RB), lambda i, j: (i, 0, 0))],
        out_shape=[jax.ShapeDtypeStruct((_GR, 1, _RB), jnp.float32),
                   jax.ShapeDtypeStruct((_GR, 1, _RB), jnp.float32)],
        compiler_params=pltpu.CompilerParams(
            dimension_semantics=("parallel", "arbitrary")),
    )(x0r, x1r)

    r0, r1, u0 = pl.pallas_call(
        _topk_body,
        out_shape=[jax.ShapeDtypeStruct((B, TOPK), jnp.int32)] * 3,
    )(s0.reshape(B, C), s1.reshape(B, C))

    grid_spec = pltpu.PrefetchScalarGridSpec(
        num_scalar_prefetch=3,
        grid=(NPLANES,),
        in_specs=[
            pl.BlockSpec((1, 1, HW), lambda i, r0s, r1s, u0s: (r0s[i], 0, 0)),
            pl.BlockSpec((1, 1, HW), lambda i, r0s, r1s, u0s: (r1s[i], 0, 0)),
        ],
        out_specs=pl.BlockSpec((1, 1, HW), lambda i, r0s, r1s, u0s: (i, 0, 0)),
    )
    out = pl.pallas_call(
        _gather_body,
        grid_spec=grid_spec,
        out_shape=jax.ShapeDtypeStruct((NPLANES, 1, HW), jnp.float32),
    )(r0.reshape(-1), r1.reshape(-1), u0.reshape(-1),
      x0r.reshape(ROWS, 1, HW), x1r.reshape(ROWS, 1, HW))

    return out.reshape(B, TOPK, H, W)


# speedup vs baseline: 2.1362x; 2.1362x over previous
import jax
import jax.numpy as jnp
from jax import lax
from jax.experimental import pallas as pl
from jax.experimental.pallas import tpu as pltpu

B, C, H, W = 8, 192, 224, 224
HW = H * W
ROWS = B * C
C2 = 2 * C
TOPK = 64
NPLANES = B * TOPK

_RB = 128
_CB = 7168
_GR = ROWS // _RB
_GC = HW // _CB


def _pool_body(x0_ref, x1_ref, s0_ref, s1_ref):
    j = pl.program_id(1)

    @pl.when(j == 0)
    def _():
        s0_ref[...] = jnp.zeros_like(s0_ref)
        s1_ref[...] = jnp.zeros_like(s1_ref)

    s0_ref[0, 0, :] += jnp.sum(x0_ref[...], axis=1)
    s1_ref[0, 0, :] += jnp.sum(x1_ref[...], axis=1)


def _topk_body(s0_ref, s1_ref, r0_ref, r1_ref, u0_ref):
    vals = jnp.concatenate([s0_ref[...], s1_ref[...]], axis=1) / float(HW)
    iota_c = lax.broadcasted_iota(jnp.int32, (B, C2), 1)
    iota_k = lax.broadcasted_iota(jnp.int32, (B, TOPK), 1)
    idxm = jnp.zeros((B, TOPK), jnp.int32)
    for k in range(TOPK):
        m = jnp.max(vals, axis=1, keepdims=True)
        cand = jnp.where(vals == m, iota_c, jnp.int32(2**30))
        sel = jnp.min(cand, axis=1)
        idxm = jnp.where(iota_k == k, sel[:, None], idxm)
        vals = jnp.where(iota_c == sel[:, None], -jnp.inf, vals)
    rowbase = lax.broadcasted_iota(jnp.int32, (B, TOPK), 0) * C
    r0_ref[...] = rowbase + jnp.minimum(idxm, C - 1)
    r1_ref[...] = rowbase + jnp.maximum(idxm - C, 0)
    u0_ref[...] = (idxm < C).astype(jnp.int32)


def _gather_body(r0s, r1s, u0s, a_ref, b_ref, o_ref):
    del r0s, r1s
    i = pl.program_id(0)
    o_ref[...] = jnp.where(u0s[i] == 1, a_ref[...], b_ref[...])


def kernel(x_0, x_1):
    x0r = x_0.reshape(ROWS, HW)
    x1r = x_1.reshape(ROWS, HW)

    s0, s1 = pl.pallas_call(
        _pool_body,
        grid=(_GR, _GC),
        in_specs=[pl.BlockSpec((_RB, _CB), lambda i, j: (i, j)),
                  pl.BlockSpec((_RB, _CB), lambda i, j: (i, j))],
        out_specs=[pl.BlockSpec((1, 1, _RB), lambda i, j: (i, 0, 0)),
                   pl.BlockSpec((1, 1, _RB), lambda i, j: (i, 0, 0))],
        out_shape=[jax.ShapeDtypeStruct((_GR, 1, _RB), jnp.float32),
                   jax.ShapeDtypeStruct((_GR, 1, _RB), jnp.float32)],
        compiler_params=pltpu.CompilerParams(
            dimension_semantics=("parallel", "arbitrary")),
    )(x0r, x1r)

    r0, r1, u0 = pl.pallas_call(
        _topk_body,
        out_shape=[jax.ShapeDtypeStruct((B, TOPK), jnp.int32)] * 3,
    )(s0.reshape(B, C), s1.reshape(B, C))

    grid_spec = pltpu.PrefetchScalarGridSpec(
        num_scalar_prefetch=3,
        grid=(NPLANES,),
        in_specs=[
            pl.BlockSpec((1, HW // 128, 128),
                         lambda i, r0s, r1s, u0s: (r0s[i], 0, 0)),
            pl.BlockSpec((1, HW // 128, 128),
                         lambda i, r0s, r1s, u0s: (r1s[i], 0, 0)),
        ],
        out_specs=pl.BlockSpec((1, HW // 128, 128),
                               lambda i, r0s, r1s, u0s: (i, 0, 0)),
    )
    out = pl.pallas_call(
        _gather_body,
        grid_spec=grid_spec,
        out_shape=jax.ShapeDtypeStruct((NPLANES, HW // 128, 128), jnp.float32),
    )(r0.reshape(-1), r1.reshape(-1), u0.reshape(-1),
      x0r.reshape(ROWS, HW // 128, 128), x1r.reshape(ROWS, HW // 128, 128))

    return out.reshape(B, TOPK, H, W)
